# Initial kernel scaffold; baseline (speedup 1.0000x reference)
import jax
import jax.numpy as jnp
from jax.experimental import pallas as pl


def kernel(x_tx, x_bd, edge_index_tt, src_tb, dst_tb, lbl_tx, lbl_bd, params):
    raise NotImplementedError("write your pallas kernel here")


# speedup vs baseline: 1.0681x; 1.0681x over previous
import functools

import jax
import jax.numpy as jnp
from jax import lax
from jax.experimental import pallas as pl
from jax.experimental.pallas import tpu as pltpu
from jax.experimental.pallas import tpu_sc as plsc

_NC = 2
_NS = 16
_NW = _NC * _NS

_F32 = jnp.float32


def _prelu(x, a):
    return jnp.where(x >= 0, x, a * x)


def _row_block(r, cap=8192):
    b = min(r, cap)
    b -= b % 8
    while b >= 8:
        if r % b == 0:
            return b
        b -= 8
    return r


def _mm(x, w, b, aff=None):
    r, k = x.shape
    c = w.shape[1]
    br = _row_block(r)
    if aff is None:
        av, bv = jnp.ones((k,), _F32), jnp.zeros((k,), _F32)
    else:
        av, bv = aff

    def body(a_ref, ab_ref, x_ref, w_ref, b_ref, o_ref):
        x2 = x_ref[...] * a_ref[...] + ab_ref[...]
        o_ref[...] = (
            jnp.dot(x2, w_ref[...], preferred_element_type=_F32)
            + b_ref[...]
        )

    return pl.pallas_call(
        body,
        grid=(r // br,),
        in_specs=[
            pl.BlockSpec((1, k), lambda i: (0, 0)),
            pl.BlockSpec((1, k), lambda i: (0, 0)),
            pl.BlockSpec((br, k), lambda i: (i, 0)),
            pl.BlockSpec((k, c), lambda i: (0, 0)),
            pl.BlockSpec((1, c), lambda i: (0, 0)),
        ],
        out_specs=pl.BlockSpec((br, c), lambda i: (i, 0)),
        out_shape=jax.ShapeDtypeStruct((r, c), _F32),
    )(av.reshape(1, k), bv.reshape(1, k), x, w, b.reshape(1, c))


def _edge_mlp(sp, wbd, b2t, scal):
    e4 = sp.shape[0]
    br = _row_block(e4)

    def body(s_ref, x_ref, w_ref, b_ref, o_ref):
        m1 = _prelu(x_ref[...], s_ref[0])
        o_ref[...] = _prelu(
            jnp.dot(m1, w_ref[...], preferred_element_type=_F32) + b_ref[...],
            s_ref[1],
        )

    return pl.pallas_call(
        body,
        grid=(e4 // br,),
        in_specs=[
            pl.BlockSpec(memory_space=pltpu.SMEM),
            pl.BlockSpec((br, 128), lambda i: (i, 0)),
            pl.BlockSpec((128, 128), lambda i: (0, 0)),
            pl.BlockSpec((1, 128), lambda i: (0, 0)),
        ],
        out_specs=pl.BlockSpec((br, 128), lambda i: (i, 0)),
        out_shape=jax.ShapeDtypeStruct((e4, 128), _F32),
    )(scal, sp, wbd, b2t.reshape(1, 128))


def _node_update(t, agg, lane_p, wa, w2, b2, scal, n1, n_real):
    c = w2.shape[1]
    br = _row_block(n1)

    def body(s_ref, t_ref, ag_ref, wa_ref, w2_ref, b2_ref, h_ref, st_ref):
        i = pl.program_id(0)
        p = t_ref[...][:, lane_p:lane_p + 32]
        h1 = _prelu(
            p
            + jnp.dot(ag_ref[...][:, :32], wa_ref[...],
                      preferred_element_type=_F32),
            s_ref[0],
        )
        h3 = _prelu(
            jnp.dot(h1, w2_ref[...], preferred_element_type=_F32) + b2_ref[...],
            s_ref[1],
        )
        h_ref[...] = h3
        rows = i * br + lax.broadcasted_iota(jnp.int32, (br, 1), 0)
        h3m = jnp.where(rows < n_real, h3, 0.0)
        st = jnp.concatenate(
            [
                jnp.sum(h3m, axis=0, keepdims=True),
                jnp.sum(h3m * h3m, axis=0, keepdims=True),
                jnp.zeros((6, c), _F32),
            ],
            axis=0,
        )

        @pl.when(i == 0)
        def _():
            st_ref[...] = st

        @pl.when(i > 0)
        def _():
            st_ref[...] = st_ref[...] + st

    return pl.pallas_call(
        body,
        grid=(n1 // br,),
        in_specs=[
            pl.BlockSpec(memory_space=pltpu.SMEM),
            pl.BlockSpec((br, 128), lambda i: (i, 0)),
            pl.BlockSpec((br, 128), lambda i: (i, 0)),
            pl.BlockSpec((32, 32), lambda i: (0, 0)),
            pl.BlockSpec((32, c), lambda i: (0, 0)),
            pl.BlockSpec((1, c), lambda i: (0, 0)),
        ],
        out_specs=[
            pl.BlockSpec((br, c), lambda i: (i, 0)),
            pl.BlockSpec((8, c), lambda i: (0, 0)),
        ],
        out_shape=[
            jax.ShapeDtypeStruct((n1, c), _F32),
            jax.ShapeDtypeStruct((8, c), _F32),
        ],
    )(scal, t, agg, wa, w2, b2.reshape(1, c))


def _mlp2(x, aff, w1, b1, w2, b2, scal):
    r, k = x.shape
    h = w1.shape[1]
    c = w2.shape[1]
    br = _row_block(r)
    av, bv = aff

    def body(s_ref, a_ref, ab_ref, x_ref, w1_ref, b1_ref, w2_ref, b2_ref,
             o_ref):
        x2 = x_ref[...] * a_ref[...] + ab_ref[...]
        h1 = _prelu(
            jnp.dot(x2, w1_ref[...], preferred_element_type=_F32)
            + b1_ref[...],
            s_ref[0],
        )
        o = jnp.dot(h1, w2_ref[...], preferred_element_type=_F32) + b2_ref[...]
        o_ref[...] = jnp.concatenate(
            [o, jnp.zeros((br, 128 - c), _F32)], axis=1)

    return pl.pallas_call(
        body,
        grid=(r // br,),
        in_specs=[
            pl.BlockSpec(memory_space=pltpu.SMEM),
            pl.BlockSpec((1, k), lambda i: (0, 0)),
            pl.BlockSpec((1, k), lambda i: (0, 0)),
            pl.BlockSpec((br, k), lambda i: (i, 0)),
            pl.BlockSpec((k, h), lambda i: (0, 0)),
            pl.BlockSpec((1, h), lambda i: (0, 0)),
            pl.BlockSpec((h, c), lambda i: (0, 0)),
            pl.BlockSpec((1, c), lambda i: (0, 0)),
        ],
        out_specs=pl.BlockSpec((br, 128), lambda i: (i, 0)),
        out_shape=jax.ShapeDtypeStruct((r, 128), _F32),
    )(scal, av.reshape(1, k), bv.reshape(1, k), x, w1, b1.reshape(1, h),
      w2, b2.reshape(1, c))


def _edge_head(ga, gb, w1t, w1b, b1, w2, b2, scal):
    r = ga.shape[0]
    hd = w1t.shape[1]
    br = _row_block(r)

    def body(s_ref, a_ref, b_ref, w1t_ref, w1b_ref, b1_ref, w2_ref, b2_ref,
             o_ref):
        a = _prelu(a_ref[...][:, :32], s_ref[0])
        b = _prelu(b_ref[...][:, :32], s_ref[0])
        h1 = _prelu(
            jnp.dot(a, w1t_ref[...], preferred_element_type=_F32)
            + jnp.dot(b, w1b_ref[...], preferred_element_type=_F32)
            + b1_ref[...],
            s_ref[1],
        )
        o_ref[...] = (
            jnp.dot(h1, w2_ref[...], preferred_element_type=_F32) + b2_ref[...]
        )

    return pl.pallas_call(
        body,
        grid=(r // br,),
        in_specs=[
            pl.BlockSpec(memory_space=pltpu.SMEM),
            pl.BlockSpec((br, 128), lambda i: (i, 0)),
            pl.BlockSpec((br, 128), lambda i: (i, 0)),
            pl.BlockSpec((32, hd), lambda i: (0, 0)),
            pl.BlockSpec((32, hd), lambda i: (0, 0)),
            pl.BlockSpec((1, hd), lambda i: (0, 0)),
            pl.BlockSpec((hd, 8), lambda i: (0, 0)),
            pl.BlockSpec((1, 8), lambda i: (0, 0)),
        ],
        out_specs=pl.BlockSpec((br, 8), lambda i: (i, 0)),
        out_shape=jax.ShapeDtypeStruct((r, 8), _F32),
    )(scal, ga, gb, w1t, w1b, b1, w2, b2)


def _sc_mesh():
    return plsc.VectorSubcoreMesh(core_axis_name="c", subcore_axis_name="s")


_CG = 256


def _sc_gather1(table, idx1d):
    m = idx1d.shape[0]
    per_w = m // _NW
    n_it = per_w // _CG

    @functools.partial(
        pl.kernel,
        out_type=jax.ShapeDtypeStruct((m, 128), _F32),
        mesh=_sc_mesh(),
        scratch_types=[
            pltpu.VMEM((_CG,), jnp.int32),
            pltpu.VMEM((_CG, 128), _F32),
            pltpu.SemaphoreType.DMA,
        ],
    )
    def k(tab_h, idx_h, out_h, ix_v, rows_v, sem):
        wid = lax.axis_index("s") * _NC + lax.axis_index("c")
        base = wid * per_w

        def body(j, carry):
            off = pl.multiple_of(base + j * _CG, 8)
            pltpu.sync_copy(idx_h.at[pl.ds(off, _CG)], ix_v)
            cps = [
                pltpu.async_copy(
                    tab_h.at[ix_v.at[pl.ds(q * 128, 128)]],
                    rows_v.at[pl.ds(q * 128, 128)],
                    sem,
                )
                for q in range(_CG // 128)
            ]
            for cp in cps:
                cp.wait()
            pltpu.sync_copy(rows_v, out_h.at[pl.ds(off, _CG)])
            return carry

        lax.fori_loop(0, n_it, body, 0)

    return k(table, idx1d)


def _sc_edge_combine(td, ts, dst1d, src1d, src_lane):
    e = dst1d.shape[0]
    per_w = e // _NW
    n_it = per_w // _CG

    @functools.partial(
        pl.kernel,
        out_type=jax.ShapeDtypeStruct((e // 4, 128), _F32),
        mesh=_sc_mesh(),
        scratch_types=[
            pltpu.VMEM((_CG,), jnp.int32),
            pltpu.VMEM((_CG,), jnp.int32),
            pltpu.VMEM((_CG, 128), _F32),
            pltpu.VMEM((_CG, 128), _F32),
            pltpu.VMEM((_CG // 4, 128), _F32),
            pltpu.SemaphoreType.DMA,
            pltpu.SemaphoreType.DMA,
        ],
    )
    def k(td_h, ts_h, dst_h, src_h, out_h, ixd_v, ixs_v, rd_v, rs_v, sp_v,
          semd, sems):
        wid = lax.axis_index("s") * _NC + lax.axis_index("c")
        base = wid * per_w
        base4 = wid * (per_w // 4)

        def body(j, carry):
            off = pl.multiple_of(base + j * _CG, 8)
            pltpu.sync_copy(dst_h.at[pl.ds(off, _CG)], ixd_v)
            pltpu.sync_copy(src_h.at[pl.ds(off, _CG)], ixs_v)
            cps = []
            for q in range(_CG // 128):
                sl = pl.ds(q * 128, 128)
                cps.append(pltpu.async_copy(
                    td_h.at[ixd_v.at[sl]], rd_v.at[sl], semd))
                cps.append(pltpu.async_copy(
                    ts_h.at[ixs_v.at[sl]], rs_v.at[sl], sems))
            for cp in cps:
                cp.wait()

            def comb(r, carry2):
                for kk in range(4):
                    ee = r * 4 + kk
                    for hh in range(2):
                        sp_v[r, pl.ds(kk * 32 + hh * 16, 16)] = (
                            rd_v[ee, pl.ds(hh * 16, 16)]
                            + rs_v[ee, pl.ds(src_lane + hh * 16, 16)]
                        )
                return carry2

            lax.fori_loop(0, _CG // 4, comb, 0)
            off4 = pl.multiple_of(base4 + j * (_CG // 4), 8)
            pltpu.sync_copy(sp_v, out_h.at[pl.ds(off4, _CG // 4)])
            return carry

        lax.fori_loop(0, n_it, body, 0)

    return k(td, ts, dst1d, src1d)


def _sc_scatter_add(m2p, dst2d, n_rows, n_it, r_rows, n_pass):
    n_blk = r_rows // 128
    blk_per_s = (n_blk + _NS - 1) // _NS
    zt_blk = (r_rows + 128) // 128
    zt_per_s = (zt_blk + _NS - 1) // _NS

    @functools.partial(
        pl.kernel,
        out_type=jax.ShapeDtypeStruct((n_rows, 128), _F32),
        mesh=_sc_mesh(),
        scratch_types=[
            pltpu.VMEM((8, 128), jnp.int32),
            pltpu.VMEM((8, 128), jnp.int32),
            pltpu.VMEM((2, 32, 128), _F32),
            pltpu.VMEM((128, 128), _F32),
            pltpu.VMEM_SHARED((r_rows + 128, 128), _F32),
            pltpu.SemaphoreType.DMA,
            pltpu.SemaphoreType.DMA,
        ],
    )
    def k(m2_h, dst_h, out_h, ix_v, ix2_v, pk_v, sc_v, table, semp, sems):
        c = lax.axis_index("c")
        s = lax.axis_index("s")
        for p in range(n_pass):
            base = (2 * p + c) * r_rows

            def zb(rr, carry):
                for ll in range(8):
                    sc_v[rr, pl.ds(ll * 16, 16)] = jnp.zeros((16,), _F32)
                return carry

            lax.fori_loop(0, 128, zb, 0)

            def zt(t, carry):
                blk = s + t * _NS

                @pl.when(blk < zt_blk)
                def _():
                    off = pl.multiple_of(blk * 128, 8)
                    pltpu.sync_copy(sc_v, table.at[pl.ds(off, 128)])

                return carry

            lax.fori_loop(0, zt_per_s, zt, 0)
            plsc.subcore_barrier()

            def body(j, carry):
                eoff = s * (n_it * 1024) + j * 1024
                ioff = pl.multiple_of(eoff // 128, 8)
                poff = pl.multiple_of(eoff // 4, 8)
                pltpu.sync_copy(dst_h.at[pl.ds(ioff, 8)], ix_v)
                for q in range(8):
                    for ll in range(8):
                        sl = pl.ds(ll * 16, 16)
                        v = ix_v[q, sl] - base
                        ok = (v >= 0) & (v < r_rows)
                        ix2_v[q, sl] = jnp.where(ok, v, r_rows)
                pltpu.sync_copy(m2_h.at[pl.ds(poff, 32)], pk_v.at[0])
                cps = [None] * 8
                pkc = [None] * 8
                for q in range(8):
                    if q < 7:
                        pkc[q + 1] = pltpu.async_copy(
                            m2_h.at[pl.ds(poff + (q + 1) * 32, 32)],
                            pk_v.at[(q + 1) % 2], semp)
                    if q > 0:
                        pkc[q].wait()
                        cps[q - 1].wait()
                    b = q % 2

                    def ex(r, carry2):
                        for kk in range(4):
                            for hh in range(2):
                                sc_v[r * 4 + kk, pl.ds(hh * 16, 16)] = (
                                    pk_v[b, r, pl.ds(kk * 32 + hh * 16, 16)])
                        return carry2

                    lax.fori_loop(0, 32, ex, 0)
                    cps[q] = pltpu.async_copy(
                        sc_v, table.at[ix2_v.at[q]], sems, add=True)
                cps[7].wait()
                return carry

            lax.fori_loop(0, n_it, body, 0)
            plsc.subcore_barrier()

            def fb(t, carry):
                blk = s + t * _NS

                @pl.when(blk < n_blk)
                def _():
                    off = pl.multiple_of(blk * 128, 8)
                    pltpu.sync_copy(table.at[pl.ds(off, 128)], sc_v)
                    off2 = pl.multiple_of(base + off, 8)
                    pltpu.sync_copy(sc_v, out_h.at[pl.ds(off2, 128)])

                return carry

            lax.fori_loop(0, blk_per_s, fb, 0)
            plsc.subcore_barrier()

    return k(m2p, dst2d)


_N_TX = 100000
_N_BD = 10000
_N_TX1 = 100352
_N_BD1 = 10240
_E_TT = 1600000
_E_TT1 = 1638400
_E_TB = 400000
_E_TB1 = 425984
_N_LBL = 100000
_N_LBL1 = 131072


def _pad1(x, n, val):
    return jnp.concatenate([x, jnp.full((n - x.shape[0],), val, x.dtype)])


def _pad_rows(x, n):
    return jnp.concatenate(
        [x, jnp.zeros((n - x.shape[0], x.shape[1]), x.dtype)], axis=0)


def _bn_affine(st, n, g, b):
    s = st[0]
    ss = st[1]
    mean = s / n
    var = ss / n - mean * mean
    a = g * lax.rsqrt(var + 1e-5)
    return a, b - mean * a


def _zeros32():
    return jnp.zeros((32,), _F32)


def _tx_table_weights(p_tt, p_tb, d):
    return (jnp.concatenate([p_tt['e_W1'][:d], p_tt['e_W1'][d:],
                             p_tt['n_W1'][:d], p_tb['e_W1'][d:]], axis=1),
            jnp.concatenate([p_tt['e_b1'], _zeros32(), p_tt['n_b1'],
                             _zeros32()]))


def _bd_table_weights(p_tb, d):
    z = jnp.zeros((d, 64), _F32)
    return (jnp.concatenate([p_tb['e_W1'][:d], p_tb['n_W1'][:d], z], axis=1),
            jnp.concatenate([p_tb['e_b1'], p_tb['n_b1'], _zeros32(),
                             _zeros32()]))


def _edge_block(p, t_dst, t_src, dst1d, src1d, dst2d, src_lane, n_rows,
                s_cfg):
    sp = _sc_edge_combine(t_dst, t_src, dst1d, src1d, src_lane)
    wbd = jnp.kron(jnp.eye(4, dtype=_F32), p['e_W2'])
    m2p = _edge_mlp(sp, wbd, jnp.tile(p['e_b2'], 4),
                    jnp.stack([p['e_a1'], p['e_a2']]))
    return _sc_scatter_add(m2p, dst2d, n_rows, s_cfg[0], s_cfg[1], s_cfg[2])


def kernel(x_tx, x_bd, edge_index_tt, src_tb, dst_tb, lbl_tx, lbl_bd, params):
    prm = params

    i32 = jnp.int32
    xtx1d = _pad1(x_tx.astype(i32), _N_LBL1, 0)
    src_tt1 = _pad1(edge_index_tt[0].astype(i32), _E_TT1, 0)
    dst_tt1 = _pad1(edge_index_tt[1].astype(i32), _E_TT1, _N_TX)
    src_tb1 = _pad1(src_tb.astype(i32), _E_TB1, 0)
    dst_tb1 = _pad1(dst_tb.astype(i32), _E_TB1, _N_BD)
    lblt1 = _pad1(lbl_tx.astype(i32), _N_LBL1, 0)
    lblb1 = _pad1(lbl_bd.astype(i32), _N_LBL1, 0)
    dst_tt2 = dst_tt1.reshape(-1, 128)
    dst_tb2 = dst_tb1.reshape(-1, 128)

    tt_s = (100, 12544, 4)
    tb_s = (26, 5120, 1)

    d1, d2 = 16, 32
    c1t, c1b = prm['c1_tt'], prm['c1_tb']
    c2t, c2b = prm['c2_tt'], prm['c2_tb']

    w_tx1, b_tx1 = _tx_table_weights(c1t, c1b, d1)
    embt = _mm(_pad_rows(prm['emb_tx'], 504), w_tx1, b_tx1)
    t_tx1 = _sc_gather1(embt, xtx1d)
    w_bd1, b_bd1 = _bd_table_weights(c1b, d1)
    bd0 = _mm(_pad_rows(x_bd, _N_BD1), prm['bd_W'], prm['bd_b'])
    t_bd1 = _mm(bd0, w_bd1, b_bd1)

    agg = _edge_block(c1t, t_tx1, t_tx1, dst_tt1, src_tt1, dst_tt2,
                      32, _N_TX1, tt_s)
    tx1r, st1t = _node_update(t_tx1, agg, 64, c1t['n_W1'][d1:],
                              c1t['n_W2'], c1t['n_b2'],
                              jnp.stack([c1t['n_a1'], c1t['act_a']]),
                              _N_TX1, _N_TX)
    agg = _edge_block(c1b, t_bd1, t_tx1, dst_tb1, src_tb1, dst_tb2,
                      96, _N_BD1, tb_s)
    bd1r, st1b = _node_update(t_bd1, agg, 32, c1b['n_W1'][d1:],
                              c1b['n_W2'], c1b['n_b2'],
                              jnp.stack([c1b['n_a1'], c1b['act_a']]),
                              _N_BD1, _N_BD)
    a1t = _bn_affine(st1t, _N_TX, c1t['bn_g'], c1t['bn_b'])
    a1b = _bn_affine(st1b, _N_BD, c1b['bn_g'], c1b['bn_b'])

    w_tx2, b_tx2 = _tx_table_weights(c2t, c2b, d2)
    t_tx2 = _mm(tx1r, w_tx2, b_tx2, aff=a1t)
    w_bd2, b_bd2 = _bd_table_weights(c2b, d2)
    t_bd2 = _mm(bd1r, w_bd2, b_bd2, aff=a1b)

    agg = _edge_block(c2t, t_tx2, t_tx2, dst_tt1, src_tt1, dst_tt2,
                      32, _N_TX1, tt_s)
    tx2r, st2t = _node_update(t_tx2, agg, 64, c2t['n_W1'][d2:],
                              c2t['n_W2'], c2t['n_b2'],
                              jnp.stack([c2t['n_a1'], c2t['act_a']]),
                              _N_TX1, _N_TX)
    agg = _edge_block(c2b, t_bd2, t_tx2, dst_tb1, src_tb1, dst_tb2,
                      96, _N_BD1, tb_s)
    bd2r, st2b = _node_update(t_bd2, agg, 32, c2b['n_W1'][d2:],
                              c2b['n_W2'], c2b['n_b2'],
                              jnp.stack([c2b['n_a1'], c2b['act_a']]),
                              _N_BD1, _N_BD)
    a2t = _bn_affine(st2t, _N_TX, c2t['bn_g'], c2t['bn_b'])
    a2b = _bn_affine(st2b, _N_BD, c2b['bn_g'], c2b['bn_b'])

    mt = prm['mlp_tx']
    tx_out = _mlp2(tx2r, a2t, mt['W1'], mt['b1'], mt['W2'], mt['b2'],
                   jnp.stack([mt['a1']]))
    mb = prm['mlp_bd']
    bd_out = _mlp2(bd2r, a2b, mb['W1'], mb['b1'], mb['W2'], mb['b2'],
                   jnp.stack([mb['a1']]))

    ga = _sc_gather1(tx_out, lblt1)
    gb = _sc_gather1(bd_out, lblb1)
    pe = prm['edge']
    hd = pe['W1'].shape[1]
    w2p = jnp.concatenate([pe['W2'], jnp.zeros((hd, 7), _F32)], axis=1)
    b2p = jnp.concatenate([pe['b2'], jnp.zeros((7,), _F32)]).reshape(1, 8)
    e = _edge_head(ga, gb, pe['W1'][:32], pe['W1'][32:],
                   pe['b1'].reshape(1, hd), w2p, b2p,
                   jnp.stack([pe['a0'], pe['a1']]))

    return (tx_out[:_N_TX, :32], bd_out[:_N_BD, :32], e[:_N_LBL, 0])
